# baseline (speedup 1.0000x reference)
import functools

import jax
import jax.numpy as jnp
from jax import lax
from jax.experimental import pallas as pl
from jax.experimental.pallas import tpu as pltpu
from jax.experimental.pallas import tpu_sc as plsc

_NC = 2
_NS = 16
_LANES = 16


def _dyn_slice(ref, start, blk):
    wsl = blk + 128
    fine = lax.rem(start, 128)
    coarse = pl.multiple_of(start - fine, 128)
    win = ref[:, pl.ds(coarse, wsl)]
    rolled = pltpu.roll(win, lax.rem(wsl - fine, wsl), axis=1)
    return rolled[:, :blk]


def _select_block(q, bank_blk, ext, blk, size, bs):
    col = lax.broadcasted_iota(jnp.int32, (1, blk), 1)
    off = q + col
    wrapped = off >= size
    offm = jnp.where(wrapped, off - size, off)
    mask = offm < bs
    start1 = blk + jnp.minimum(q, bs)
    start2 = jnp.clip(q - (size - blk), 0, blk)
    part1 = _dyn_slice(ext, start1, blk)
    part2 = _dyn_slice(ext, start2, blk)
    g = jnp.where(wrapped, part2, part1)
    return jnp.where(mask, g, bank_blk)


def _tc_body(ptr_ref, bank_ref, ext_ref, pre_ref, new_ref,
             ext_v, in_v, out_v, sems, hsems,
             dense_v, dsem, dsem_pre, dsem_new,
             *, blk, size, bs, nhit, nbuf):
    p = ptr_ref[0]
    nb = size // blk

    cp_ext = pltpu.make_async_copy(ext_ref, ext_v, sems.at[0])
    cp_ext.start()

    k0 = lax.div(p, blk)
    hits = []
    for h in range(nhit):
        k = lax.rem(k0 + h, nb)
        s = pl.multiple_of(k * blk, blk)
        ld = pltpu.make_async_copy(bank_ref.at[:, pl.ds(s, blk)],
                                   in_v.at[h], hsems.at[h])
        ld.start()
        hits.append((k, s, ld))

    def mk_load(c):
        return pltpu.make_async_copy(bank_ref.at[:, pl.ds(c * blk, blk)],
                                     dense_v.at[c % nbuf], dsem.at[c % nbuf])

    def mk_store(c, dst_ref, sem):
        return pltpu.make_async_copy(dense_v.at[c % nbuf],
                                     dst_ref.at[:, pl.ds(c * blk, blk)],
                                     sem.at[c % nbuf])

    lag = nbuf - 2
    for c in range(nb):
        if c >= nbuf:
            mk_store(c - nbuf, pre_ref, dsem_pre).wait()
            mk_store(c - nbuf, new_ref, dsem_new).wait()
        mk_load(c).start()
        if c >= lag:
            d = c - lag
            mk_load(d).wait()
            mk_store(d, pre_ref, dsem_pre).start()
            mk_store(d, new_ref, dsem_new).start()
    for d in range(nb - lag, nb):
        mk_load(d).wait()
        mk_store(d, pre_ref, dsem_pre).start()
        mk_store(d, new_ref, dsem_new).start()
    dense_tail = min(nbuf, nb)

    cp_ext.wait()
    for h, (k, s, ld) in enumerate(hits):
        ld.wait()
        q = lax.rem(k * blk - p + size, size)
        out_v[h] = _select_block(q, in_v[h], ext_v, blk, size, bs)

    for d in range(nb - dense_tail, nb):
        mk_store(d, pre_ref, dsem_pre).wait()
        mk_store(d, new_ref, dsem_new).wait()
    stores = []
    for h, (k, s, ld) in enumerate(hits):
        st = pltpu.make_async_copy(out_v.at[h], new_ref.at[:, pl.ds(s, blk)],
                                   hsems.at[h])
        st.start()
        stores.append(st)
    for st in stores:
        st.wait()


def _tc_bank(output, bank, ptr_arr):
    dim, size = bank.shape
    bs = output.shape[0]
    blk = 4096
    nhit = (bs - 1) // blk + 2
    w = bs + 2 * blk + 128
    nbuf = 8

    out_t = output.T.astype(jnp.float32)
    zpad = jnp.zeros((dim, blk), jnp.float32)
    zpad_r = jnp.zeros((dim, blk + 128), jnp.float32)
    ext = jnp.concatenate([zpad, out_t, zpad_r], axis=1)

    out_shapes = (
        jax.ShapeDtypeStruct((dim, size), jnp.float32),
        jax.ShapeDtypeStruct((dim, size), jnp.float32),
    )
    any_spec = pl.BlockSpec(memory_space=pl.ANY)
    fn = pl.pallas_call(
        functools.partial(_tc_body, blk=blk, size=size, bs=bs, nhit=nhit,
                          nbuf=nbuf),
        in_specs=[
            pl.BlockSpec(memory_space=pltpu.SMEM),
            any_spec, any_spec,
        ],
        out_specs=[any_spec, any_spec],
        out_shape=out_shapes,
        scratch_shapes=[
            pltpu.VMEM((dim, w), jnp.float32),
            pltpu.VMEM((nhit, dim, blk), jnp.float32),
            pltpu.VMEM((nhit, dim, blk), jnp.float32),
            pltpu.SemaphoreType.DMA((2,)),
            pltpu.SemaphoreType.DMA((2 * nhit,)),
            pltpu.VMEM((nbuf, dim, blk), jnp.float32),
            pltpu.SemaphoreType.DMA((nbuf,)),
            pltpu.SemaphoreType.DMA((nbuf,)),
            pltpu.SemaphoreType.DMA((nbuf,)),
        ],
    )
    return fn(ptr_arr, bank, ext)


def _sc_labels_body(ptr_hbm, bl_hbm, lab_hbm, lpre_hbm, lnew_hbm,
                    buf_v, lab_v, ptr_v, *, size, bs):
    nw = _NC * _NS
    sl = size // nw
    wid = lax.axis_index("s") * _NC + lax.axis_index("c")
    base = wid * sl

    pltpu.sync_copy(ptr_hbm, ptr_v)
    pltpu.sync_copy(lab_hbm, lab_v)
    pltpu.sync_copy(bl_hbm.at[pl.ds(base, sl)], buf_v)
    pltpu.sync_copy(buf_v, lpre_hbm.at[pl.ds(base, sl)])

    p_vec = ptr_v[...]
    base_vec = jnp.full((_LANES,), base, jnp.int32)

    def step(t, carry):
        i_vec = lax.iota(jnp.int32, _LANES) + t * _LANES
        d = lax.rem(p_vec + i_vec, size)
        local = d - base_vec
        m = jnp.logical_and(local >= 0, local < sl)
        vals = lab_v[pl.ds(t * _LANES, _LANES)]
        plsc.store_scatter(buf_v, [jnp.clip(local, 0, sl - 1)], vals, mask=m)
        return carry

    lax.fori_loop(0, bs // _LANES, step, 0)
    pltpu.sync_copy(buf_v, lnew_hbm.at[pl.ds(base, sl)])


def _sc_labels(labels, bank_labels, ptr_arr, size, bs):
    mesh = plsc.VectorSubcoreMesh(core_axis_name="c", subcore_axis_name="s",
                                  num_cores=_NC, num_subcores=_NS)
    sl = size // (_NC * _NS)
    fn = pl.kernel(
        functools.partial(_sc_labels_body, size=size, bs=bs),
        out_type=(
            jax.ShapeDtypeStruct((size,), jnp.int32),
            jax.ShapeDtypeStruct((size,), jnp.int32),
        ),
        mesh=mesh,
        compiler_params=pltpu.CompilerParams(needs_layout_passes=False),
        scratch_types=[
            pltpu.VMEM((sl,), jnp.int32),
            pltpu.VMEM((bs,), jnp.int32),
            pltpu.VMEM((_LANES,), jnp.int32),
        ],
    )
    ptr16 = jnp.broadcast_to(ptr_arr, (_LANES,))
    return fn(ptr16, bank_labels, labels)


def kernel(output, labels, bank, bank_labels, ptr):
    dim, size = bank.shape
    bs = output.shape[0]
    ptr_arr = jnp.asarray(ptr, jnp.int32).reshape(1)

    pre, new = _tc_bank(output, bank, ptr_arr)
    lpre, lnew = _sc_labels(labels, bank_labels, ptr_arr, size, bs)
    return (output, pre, lpre, new, lnew)


# speedup vs baseline: 1.0019x; 1.0019x over previous
import functools

import jax
import jax.numpy as jnp
from jax import lax
from jax.experimental import pallas as pl
from jax.experimental.pallas import tpu as pltpu
from jax.experimental.pallas import tpu_sc as plsc

_NC = 2
_NS = 16
_LANES = 16


def _dyn_slice(ref, start, blk):
    wsl = blk + 128
    fine = lax.rem(start, 128)
    coarse = pl.multiple_of(start - fine, 128)
    win = ref[:, pl.ds(coarse, wsl)]
    rolled = pltpu.roll(win, lax.rem(wsl - fine, wsl), axis=1)
    return rolled[:, :blk]


def _select_block(q, bank_blk, ext, blk, size, bs):
    col = lax.broadcasted_iota(jnp.int32, (1, blk), 1)
    off = q + col
    wrapped = off >= size
    offm = jnp.where(wrapped, off - size, off)
    mask = offm < bs
    start1 = blk + jnp.minimum(q, bs)
    start2 = jnp.clip(q - (size - blk), 0, blk)
    part1 = _dyn_slice(ext, start1, blk)
    part2 = _dyn_slice(ext, start2, blk)
    g = jnp.where(wrapped, part2, part1)
    return jnp.where(mask, g, bank_blk)


def _tc_body(ptr_ref, bank_ref, ext_ref, pre_ref, new_ref,
             ext_v, in_v, out_v, sems, hsems,
             dense_v, dsem, dsem_pre, dsem_new,
             *, blk, size, bs, nhit, nbuf):
    p = ptr_ref[0]
    nb = size // blk

    cp_ext = pltpu.make_async_copy(ext_ref, ext_v, sems.at[0])
    cp_ext.start()

    k0 = lax.div(p, blk)
    hits = []
    for h in range(nhit):
        k = lax.rem(k0 + h, nb)
        s = pl.multiple_of(k * blk, blk)
        ld = pltpu.make_async_copy(bank_ref.at[:, pl.ds(s, blk)],
                                   in_v.at[h], hsems.at[h])
        ld.start()
        hits.append((k, s, ld))

    def mk_load(c):
        return pltpu.make_async_copy(bank_ref.at[:, pl.ds(c * blk, blk)],
                                     dense_v.at[c % nbuf], dsem.at[c % nbuf])

    def mk_store(c, dst_ref, sem):
        return pltpu.make_async_copy(dense_v.at[c % nbuf],
                                     dst_ref.at[:, pl.ds(c * blk, blk)],
                                     sem.at[c % nbuf])

    lag = nbuf - 2
    for c in range(nb):
        if c >= nbuf:
            mk_store(c - nbuf, pre_ref, dsem_pre).wait()
            mk_store(c - nbuf, new_ref, dsem_new).wait()
        mk_load(c).start()
        if c >= lag:
            d = c - lag
            mk_load(d).wait()
            mk_store(d, pre_ref, dsem_pre).start()
            mk_store(d, new_ref, dsem_new).start()
    for d in range(nb - lag, nb):
        mk_load(d).wait()
        mk_store(d, pre_ref, dsem_pre).start()
        mk_store(d, new_ref, dsem_new).start()
    dense_tail = min(nbuf, nb)

    cp_ext.wait()
    for h, (k, s, ld) in enumerate(hits):
        ld.wait()
        q = lax.rem(k * blk - p + size, size)
        out_v[h] = _select_block(q, in_v[h], ext_v, blk, size, bs)

    for d in range(nb - dense_tail, nb):
        mk_store(d, pre_ref, dsem_pre).wait()
        mk_store(d, new_ref, dsem_new).wait()
    stores = []
    for h, (k, s, ld) in enumerate(hits):
        st = pltpu.make_async_copy(out_v.at[h], new_ref.at[:, pl.ds(s, blk)],
                                   hsems.at[h])
        st.start()
        stores.append(st)
    for st in stores:
        st.wait()


def _tc_bank(output, bank, ptr_arr):
    dim, size = bank.shape
    bs = output.shape[0]
    blk = 4096
    nhit = (bs - 1) // blk + 2
    w = bs + 2 * blk + 128
    nbuf = 8

    out_t = output.T.astype(jnp.float32)
    zpad = jnp.zeros((dim, blk), jnp.float32)
    zpad_r = jnp.zeros((dim, blk + 128), jnp.float32)
    ext = jnp.concatenate([zpad, out_t, zpad_r], axis=1)

    out_shapes = (
        jax.ShapeDtypeStruct((dim, size), jnp.float32),
        jax.ShapeDtypeStruct((dim, size), jnp.float32),
    )
    any_spec = pl.BlockSpec(memory_space=pl.ANY)
    fn = pl.pallas_call(
        functools.partial(_tc_body, blk=blk, size=size, bs=bs, nhit=nhit,
                          nbuf=nbuf),
        in_specs=[
            pl.BlockSpec(memory_space=pltpu.SMEM),
            any_spec, any_spec,
        ],
        out_specs=[any_spec, any_spec],
        out_shape=out_shapes,
        scratch_shapes=[
            pltpu.VMEM((dim, w), jnp.float32),
            pltpu.VMEM((nhit, dim, blk), jnp.float32),
            pltpu.VMEM((nhit, dim, blk), jnp.float32),
            pltpu.SemaphoreType.DMA((2,)),
            pltpu.SemaphoreType.DMA((2 * nhit,)),
            pltpu.VMEM((nbuf, dim, blk), jnp.float32),
            pltpu.SemaphoreType.DMA((nbuf,)),
            pltpu.SemaphoreType.DMA((nbuf,)),
            pltpu.SemaphoreType.DMA((nbuf,)),
        ],
    )
    return fn(ptr_arr, bank, ext)


def _sc_labels_body(ptr_hbm, bl_hbm, lab_hbm, lpre_hbm, lnew_hbm,
                    buf_v, lab_v, ptr_v, *, size, bs):
    nw = _NC * _NS
    sl = size // nw
    wid = lax.axis_index("s") * _NC + lax.axis_index("c")
    base = wid * sl

    pltpu.sync_copy(ptr_hbm, ptr_v)
    pltpu.sync_copy(lab_hbm, lab_v)
    pltpu.sync_copy(bl_hbm.at[pl.ds(base, sl)], buf_v)
    pltpu.sync_copy(buf_v, lpre_hbm.at[pl.ds(base, sl)])

    p_vec = ptr_v[...]
    base_vec = jnp.full((_LANES,), base, jnp.int32)

    def step(t, carry):
        i_vec = lax.iota(jnp.int32, _LANES) + t * _LANES
        d = lax.rem(p_vec + i_vec, size)
        local = d - base_vec
        m = jnp.logical_and(local >= 0, local < sl)
        vals = lab_v[pl.ds(t * _LANES, _LANES)]
        plsc.store_scatter(buf_v, [jnp.clip(local, 0, sl - 1)], vals, mask=m)
        return carry

    p = lax.reduce_max(p_vec, (0,))
    r1lo = jnp.clip(base - p, 0, bs)
    r1hi = jnp.clip(base + sl - p, 0, bs)
    r2lo = jnp.clip(base + size - p, 0, bs)
    r2hi = jnp.clip(base + sl + size - p, 0, bs)
    lax.fori_loop(lax.div(r1lo, _LANES),
                  lax.div(r1hi + _LANES - 1, _LANES), step, 0)
    lax.fori_loop(lax.div(r2lo, _LANES),
                  lax.div(r2hi + _LANES - 1, _LANES), step, 0)
    pltpu.sync_copy(buf_v, lnew_hbm.at[pl.ds(base, sl)])


def _sc_labels(labels, bank_labels, ptr_arr, size, bs):
    mesh = plsc.VectorSubcoreMesh(core_axis_name="c", subcore_axis_name="s",
                                  num_cores=_NC, num_subcores=_NS)
    sl = size // (_NC * _NS)
    fn = pl.kernel(
        functools.partial(_sc_labels_body, size=size, bs=bs),
        out_type=(
            jax.ShapeDtypeStruct((size,), jnp.int32),
            jax.ShapeDtypeStruct((size,), jnp.int32),
        ),
        mesh=mesh,
        compiler_params=pltpu.CompilerParams(needs_layout_passes=False),
        scratch_types=[
            pltpu.VMEM((sl,), jnp.int32),
            pltpu.VMEM((bs,), jnp.int32),
            pltpu.VMEM((_LANES,), jnp.int32),
        ],
    )
    ptr16 = jnp.broadcast_to(ptr_arr, (_LANES,))
    return fn(ptr16, bank_labels, labels)


def kernel(output, labels, bank, bank_labels, ptr):
    dim, size = bank.shape
    bs = output.shape[0]
    ptr_arr = jnp.asarray(ptr, jnp.int32).reshape(1)

    lpre, lnew = _sc_labels(labels, bank_labels, ptr_arr, size, bs)
    pre, new = _tc_bank(output, bank, ptr_arr)
    return (output, pre, lpre, new, lnew)


# speedup vs baseline: 1.0142x; 1.0122x over previous
import functools

import jax
import jax.numpy as jnp
from jax import lax
from jax.experimental import pallas as pl
from jax.experimental.pallas import tpu as pltpu
from jax.experimental.pallas import tpu_sc as plsc

_NC = 1
_NS = 16
_LANES = 16


def _dyn_slice(ref, start, blk):
    wsl = blk + 128
    fine = lax.rem(start, 128)
    coarse = pl.multiple_of(start - fine, 128)
    win = ref[:, pl.ds(coarse, wsl)]
    rolled = pltpu.roll(win, lax.rem(wsl - fine, wsl), axis=1)
    return rolled[:, :blk]


def _select_block(q, bank_blk, ext, blk, size, bs):
    col = lax.broadcasted_iota(jnp.int32, (1, blk), 1)
    off = q + col
    wrapped = off >= size
    offm = jnp.where(wrapped, off - size, off)
    mask = offm < bs
    start1 = blk + jnp.minimum(q, bs)
    start2 = jnp.clip(q - (size - blk), 0, blk)
    part1 = _dyn_slice(ext, start1, blk)
    part2 = _dyn_slice(ext, start2, blk)
    g = jnp.where(wrapped, part2, part1)
    return jnp.where(mask, g, bank_blk)


def _tc_body(ptr_ref, bank_ref, ext_ref, pre_ref, new_ref,
             ext_v, in_v, out_v, sems, hsems,
             dense_v, dsem, dsem_pre, dsem_new,
             *, blk, size, bs, nhit, nbuf):
    p = ptr_ref[0]
    nb = size // blk

    cp_ext = pltpu.make_async_copy(ext_ref, ext_v, sems.at[0])
    cp_ext.start()

    k0 = lax.div(p, blk)
    hits = []
    for h in range(nhit):
        k = lax.rem(k0 + h, nb)
        s = pl.multiple_of(k * blk, blk)
        ld = pltpu.make_async_copy(bank_ref.at[:, pl.ds(s, blk)],
                                   in_v.at[h], hsems.at[h])
        ld.start()
        hits.append((k, s, ld))

    def mk_load(c):
        return pltpu.make_async_copy(bank_ref.at[:, pl.ds(c * blk, blk)],
                                     dense_v.at[c % nbuf], dsem.at[c % nbuf])

    def mk_store(c, dst_ref, sem):
        return pltpu.make_async_copy(dense_v.at[c % nbuf],
                                     dst_ref.at[:, pl.ds(c * blk, blk)],
                                     sem.at[c % nbuf])

    lag = nbuf - 2
    for c in range(nb):
        if c >= nbuf:
            mk_store(c - nbuf, pre_ref, dsem_pre).wait()
            mk_store(c - nbuf, new_ref, dsem_new).wait()
        mk_load(c).start()
        if c >= lag:
            d = c - lag
            mk_load(d).wait()
            mk_store(d, pre_ref, dsem_pre).start()
            mk_store(d, new_ref, dsem_new).start()
    for d in range(nb - lag, nb):
        mk_load(d).wait()
        mk_store(d, pre_ref, dsem_pre).start()
        mk_store(d, new_ref, dsem_new).start()
    dense_tail = min(nbuf, nb)

    cp_ext.wait()
    for h, (k, s, ld) in enumerate(hits):
        ld.wait()
        q = lax.rem(k * blk - p + size, size)
        out_v[h] = _select_block(q, in_v[h], ext_v, blk, size, bs)

    for d in range(nb - dense_tail, nb):
        mk_store(d, pre_ref, dsem_pre).wait()
        mk_store(d, new_ref, dsem_new).wait()
    stores = []
    for h, (k, s, ld) in enumerate(hits):
        st = pltpu.make_async_copy(out_v.at[h], new_ref.at[:, pl.ds(s, blk)],
                                   hsems.at[h])
        st.start()
        stores.append(st)
    for st in stores:
        st.wait()


def _tc_bank(output, bank, ptr_arr):
    dim, size = bank.shape
    bs = output.shape[0]
    blk = 4096
    nhit = (bs - 1) // blk + 2
    w = bs + 2 * blk + 128
    nbuf = 8

    out_t = output.T.astype(jnp.float32)
    zpad = jnp.zeros((dim, blk), jnp.float32)
    zpad_r = jnp.zeros((dim, blk + 128), jnp.float32)
    ext = jnp.concatenate([zpad, out_t, zpad_r], axis=1)

    out_shapes = (
        jax.ShapeDtypeStruct((dim, size), jnp.float32),
        jax.ShapeDtypeStruct((dim, size), jnp.float32),
    )
    any_spec = pl.BlockSpec(memory_space=pl.ANY)
    fn = pl.pallas_call(
        functools.partial(_tc_body, blk=blk, size=size, bs=bs, nhit=nhit,
                          nbuf=nbuf),
        in_specs=[
            pl.BlockSpec(memory_space=pltpu.SMEM),
            any_spec, any_spec,
        ],
        out_specs=[any_spec, any_spec],
        out_shape=out_shapes,
        scratch_shapes=[
            pltpu.VMEM((dim, w), jnp.float32),
            pltpu.VMEM((nhit, dim, blk), jnp.float32),
            pltpu.VMEM((nhit, dim, blk), jnp.float32),
            pltpu.SemaphoreType.DMA((2,)),
            pltpu.SemaphoreType.DMA((2 * nhit,)),
            pltpu.VMEM((nbuf, dim, blk), jnp.float32),
            pltpu.SemaphoreType.DMA((nbuf,)),
            pltpu.SemaphoreType.DMA((nbuf,)),
            pltpu.SemaphoreType.DMA((nbuf,)),
        ],
    )
    return fn(ptr_arr, bank, ext)


def _sc_labels_body(ptr_hbm, bl_hbm, lab_hbm, lpre_hbm, lnew_hbm,
                    buf_v, lab_v, ptr_v, *, size, bs):
    nw = _NC * _NS
    sl = size // nw
    wid = lax.axis_index("s") * _NC + lax.axis_index("c")
    base = wid * sl

    pltpu.sync_copy(ptr_hbm, ptr_v)
    pltpu.sync_copy(lab_hbm, lab_v)
    pltpu.sync_copy(bl_hbm.at[pl.ds(base, sl)], buf_v)
    pltpu.sync_copy(buf_v, lpre_hbm.at[pl.ds(base, sl)])

    p_vec = ptr_v[...]
    base_vec = jnp.full((_LANES,), base, jnp.int32)

    def step(t, carry):
        i_vec = lax.iota(jnp.int32, _LANES) + t * _LANES
        d = lax.rem(p_vec + i_vec, size)
        local = d - base_vec
        m = jnp.logical_and(local >= 0, local < sl)
        vals = lab_v[pl.ds(t * _LANES, _LANES)]
        plsc.store_scatter(buf_v, [jnp.clip(local, 0, sl - 1)], vals, mask=m)
        return carry

    p = lax.reduce_max(p_vec, (0,))
    r1lo = jnp.clip(base - p, 0, bs)
    r1hi = jnp.clip(base + sl - p, 0, bs)
    r2lo = jnp.clip(base + size - p, 0, bs)
    r2hi = jnp.clip(base + sl + size - p, 0, bs)
    lax.fori_loop(lax.div(r1lo, _LANES),
                  lax.div(r1hi + _LANES - 1, _LANES), step, 0)
    lax.fori_loop(lax.div(r2lo, _LANES),
                  lax.div(r2hi + _LANES - 1, _LANES), step, 0)
    pltpu.sync_copy(buf_v, lnew_hbm.at[pl.ds(base, sl)])


def _sc_labels(labels, bank_labels, ptr_arr, size, bs):
    mesh = plsc.VectorSubcoreMesh(core_axis_name="c", subcore_axis_name="s",
                                  num_cores=_NC, num_subcores=_NS)
    sl = size // (_NC * _NS)
    fn = pl.kernel(
        functools.partial(_sc_labels_body, size=size, bs=bs),
        out_type=(
            jax.ShapeDtypeStruct((size,), jnp.int32),
            jax.ShapeDtypeStruct((size,), jnp.int32),
        ),
        mesh=mesh,
        compiler_params=pltpu.CompilerParams(needs_layout_passes=False),
        scratch_types=[
            pltpu.VMEM((sl,), jnp.int32),
            pltpu.VMEM((bs,), jnp.int32),
            pltpu.VMEM((_LANES,), jnp.int32),
        ],
    )
    ptr16 = jnp.broadcast_to(ptr_arr, (_LANES,))
    return fn(ptr16, bank_labels, labels)


def kernel(output, labels, bank, bank_labels, ptr):
    dim, size = bank.shape
    bs = output.shape[0]
    ptr_arr = jnp.asarray(ptr, jnp.int32).reshape(1)

    lpre, lnew = _sc_labels(labels, bank_labels, ptr_arr, size, bs)
    pre, new = _tc_bank(output, bank, ptr_arr)
    return (output, pre, lpre, new, lnew)


# speedup vs baseline: 1.0160x; 1.0018x over previous
import functools

import jax
import jax.numpy as jnp
from jax import lax
from jax.experimental import pallas as pl
from jax.experimental.pallas import tpu as pltpu
from jax.experimental.pallas import tpu_sc as plsc

_NC = 1
_NS = 16
_LANES = 16


def _dyn_slice(ref, start, blk):
    wsl = blk + 128
    fine = lax.rem(start, 128)
    coarse = pl.multiple_of(start - fine, 128)
    win = ref[:, pl.ds(coarse, wsl)]
    rolled = pltpu.roll(win, lax.rem(wsl - fine, wsl), axis=1)
    return rolled[:, :blk]


def _select_block(q, bank_blk, ext, blk, size, bs):
    col = lax.broadcasted_iota(jnp.int32, (1, blk), 1)
    off = q + col
    wrapped = off >= size
    offm = jnp.where(wrapped, off - size, off)
    mask = offm < bs
    start1 = blk + jnp.minimum(q, bs)
    start2 = jnp.clip(q - (size - blk), 0, blk)
    part1 = _dyn_slice(ext, start1, blk)
    part2 = _dyn_slice(ext, start2, blk)
    g = jnp.where(wrapped, part2, part1)
    return jnp.where(mask, g, bank_blk)


def _tc_body(ptr_ref, bank_ref, ext_ref, pre_ref, new_ref,
             ext_v, in_v, out_v, sems, hsems,
             dense_v, dsem, dsem_pre, dsem_new,
             *, blk, size, bs, nhit, nbuf):
    p = ptr_ref[0]
    nb = size // blk

    cp_ext = pltpu.make_async_copy(ext_ref, ext_v, sems.at[0])
    cp_ext.start()

    k0 = lax.div(p, blk)
    hits = []
    for h in range(nhit):
        k = lax.rem(k0 + h, nb)
        s = pl.multiple_of(k * blk, blk)
        ld = pltpu.make_async_copy(bank_ref.at[:, pl.ds(s, blk)],
                                   in_v.at[h], hsems.at[h])
        ld.start()
        hits.append((k, s, ld))

    def mk_load(c):
        return pltpu.make_async_copy(bank_ref.at[:, pl.ds(c * blk, blk)],
                                     dense_v.at[c % nbuf], dsem.at[c % nbuf])

    def mk_store(c, dst_ref, sem):
        return pltpu.make_async_copy(dense_v.at[c % nbuf],
                                     dst_ref.at[:, pl.ds(c * blk, blk)],
                                     sem.at[c % nbuf])

    lag = nbuf - 2
    for c in range(nb):
        if c >= nbuf:
            mk_store(c - nbuf, pre_ref, dsem_pre).wait()
            mk_store(c - nbuf, new_ref, dsem_new).wait()
        mk_load(c).start()
        if c >= lag:
            d = c - lag
            mk_load(d).wait()
            mk_store(d, pre_ref, dsem_pre).start()
            mk_store(d, new_ref, dsem_new).start()
    for d in range(nb - lag, nb):
        mk_load(d).wait()
        mk_store(d, pre_ref, dsem_pre).start()
        mk_store(d, new_ref, dsem_new).start()
    dense_tail = min(nbuf, nb)

    cp_ext.wait()
    for h, (k, s, ld) in enumerate(hits):
        ld.wait()
        q = lax.rem(k * blk - p + size, size)
        out_v[h] = _select_block(q, in_v[h], ext_v, blk, size, bs)

    for d in range(nb - dense_tail, nb):
        mk_store(d, pre_ref, dsem_pre).wait()
        mk_store(d, new_ref, dsem_new).wait()
    stores = []
    for h, (k, s, ld) in enumerate(hits):
        st = pltpu.make_async_copy(out_v.at[h], new_ref.at[:, pl.ds(s, blk)],
                                   hsems.at[h])
        st.start()
        stores.append(st)
    for st in stores:
        st.wait()


def _tc_bank(output, bank, ptr_arr):
    dim, size = bank.shape
    bs = output.shape[0]
    blk = 4096
    nhit = (bs - 1) // blk + 2
    w = bs + 2 * blk + 128
    nbuf = 8

    out_t = output.T.astype(jnp.float32)
    zpad = jnp.zeros((dim, blk), jnp.float32)
    zpad_r = jnp.zeros((dim, blk + 128), jnp.float32)
    ext = jnp.concatenate([zpad, out_t, zpad_r], axis=1)

    out_shapes = (
        jax.ShapeDtypeStruct((dim, size), jnp.float32),
        jax.ShapeDtypeStruct((dim, size), jnp.float32),
    )
    any_spec = pl.BlockSpec(memory_space=pl.ANY)
    fn = pl.pallas_call(
        functools.partial(_tc_body, blk=blk, size=size, bs=bs, nhit=nhit,
                          nbuf=nbuf),
        in_specs=[
            pl.BlockSpec(memory_space=pltpu.SMEM),
            any_spec, any_spec,
        ],
        out_specs=[any_spec, any_spec],
        out_shape=out_shapes,
        scratch_shapes=[
            pltpu.VMEM((dim, w), jnp.float32),
            pltpu.VMEM((nhit, dim, blk), jnp.float32),
            pltpu.VMEM((nhit, dim, blk), jnp.float32),
            pltpu.SemaphoreType.DMA((2,)),
            pltpu.SemaphoreType.DMA((2 * nhit,)),
            pltpu.VMEM((nbuf, dim, blk), jnp.float32),
            pltpu.SemaphoreType.DMA((nbuf,)),
            pltpu.SemaphoreType.DMA((nbuf,)),
            pltpu.SemaphoreType.DMA((nbuf,)),
        ],
    )
    return fn(ptr_arr, bank, ext)


def _sc_labels_body(ptr_hbm, bl_hbm, lab_hbm, lpre_hbm, lnew_hbm,
                    buf_v, lab_v, ptr_v, *, size, bs):
    nw = _NC * _NS
    sl = size // nw
    wid = lax.axis_index("s") * _NC + lax.axis_index("c")
    base = wid * sl

    pltpu.sync_copy(ptr_hbm, ptr_v)
    pltpu.sync_copy(lab_hbm, lab_v)
    pltpu.sync_copy(bl_hbm.at[pl.ds(base, sl)], buf_v)
    pltpu.sync_copy(buf_v, lpre_hbm.at[pl.ds(base, sl)])

    p_vec = ptr_v[...]
    base_vec = jnp.full((_LANES,), base, jnp.int32)

    def step(t, carry):
        i_vec = lax.iota(jnp.int32, _LANES) + t * _LANES
        d = lax.rem(p_vec + i_vec, size)
        local = d - base_vec
        m = jnp.logical_and(local >= 0, local < sl)
        vals = lab_v[pl.ds(t * _LANES, _LANES)]
        plsc.store_scatter(buf_v, [jnp.clip(local, 0, sl - 1)], vals, mask=m)
        return carry

    p = lax.reduce_max(p_vec, (0,))
    r1lo = jnp.clip(base - p, 0, bs)
    r1hi = jnp.clip(base + sl - p, 0, bs)
    r2lo = jnp.clip(base + size - p, 0, bs)
    r2hi = jnp.clip(base + sl + size - p, 0, bs)
    lax.fori_loop(lax.div(r1lo, _LANES),
                  lax.div(r1hi + _LANES - 1, _LANES), step, 0)
    lax.fori_loop(lax.div(r2lo, _LANES),
                  lax.div(r2hi + _LANES - 1, _LANES), step, 0)
    pltpu.sync_copy(buf_v, lnew_hbm.at[pl.ds(base, sl)])


def _sc_labels(labels, bank_labels, ptr_arr, size, bs):
    mesh = plsc.VectorSubcoreMesh(core_axis_name="c", subcore_axis_name="s",
                                  num_cores=_NC, num_subcores=_NS)
    sl = size // (_NC * _NS)
    fn = pl.kernel(
        functools.partial(_sc_labels_body, size=size, bs=bs),
        out_type=(
            jax.ShapeDtypeStruct((size,), jnp.int32),
            jax.ShapeDtypeStruct((size,), jnp.int32),
        ),
        mesh=mesh,
        compiler_params=pltpu.CompilerParams(needs_layout_passes=False, skip_device_barrier=True),
        scratch_types=[
            pltpu.VMEM((sl,), jnp.int32),
            pltpu.VMEM((bs,), jnp.int32),
            pltpu.VMEM((_LANES,), jnp.int32),
        ],
    )
    ptr16 = jnp.broadcast_to(ptr_arr, (_LANES,))
    return fn(ptr16, bank_labels, labels)


def kernel(output, labels, bank, bank_labels, ptr):
    dim, size = bank.shape
    bs = output.shape[0]
    ptr_arr = jnp.asarray(ptr, jnp.int32).reshape(1)

    lpre, lnew = _sc_labels(labels, bank_labels, ptr_arr, size, bs)
    pre, new = _tc_bank(output, bank, ptr_arr)
    return (output, pre, lpre, new, lnew)


# speedup vs baseline: 1.0215x; 1.0054x over previous
import functools

import jax
import jax.numpy as jnp
from jax import lax
from jax.experimental import pallas as pl
from jax.experimental.pallas import tpu as pltpu
from jax.experimental.pallas import tpu_sc as plsc

_NC = 1
_NS = 16
_LANES = 16


def _dyn_slice(ref, start, blk):
    wsl = blk + 128
    fine = lax.rem(start, 128)
    coarse = pl.multiple_of(start - fine, 128)
    win = ref[:, pl.ds(coarse, wsl)]
    rolled = pltpu.roll(win, lax.rem(wsl - fine, wsl), axis=1)
    return rolled[:, :blk]


def _select_block(q, bank_blk, ext, blk, size, bs):
    col = lax.broadcasted_iota(jnp.int32, (1, blk), 1)
    off = q + col
    wrapped = off >= size
    offm = jnp.where(wrapped, off - size, off)
    mask = offm < bs
    start1 = blk + jnp.minimum(q, bs)
    start2 = jnp.clip(q - (size - blk), 0, blk)
    part1 = _dyn_slice(ext, start1, blk)
    part2 = _dyn_slice(ext, start2, blk)
    g = jnp.where(wrapped, part2, part1)
    return jnp.where(mask, g, bank_blk)


def _tc_body(ptr_ref, bank_ref, ext_ref, pre_ref, new_ref,
             ext_v, in_v, out_v, sems, hsems,
             dense_v, dsem, dsem_pre, dsem_new,
             *, blk, size, bs, nhit, nbuf):
    p = ptr_ref[0]
    nb = size // blk

    cp_ext = pltpu.make_async_copy(ext_ref, ext_v, sems.at[0])
    cp_ext.start()

    k0 = lax.div(p, blk)
    hits = []
    for h in range(nhit):
        k = lax.rem(k0 + h, nb)
        s = pl.multiple_of(k * blk, blk)
        ld = pltpu.make_async_copy(bank_ref.at[:, pl.ds(s, blk)],
                                   in_v.at[h], hsems.at[h])
        ld.start()
        hits.append((k, s, ld))

    def mk_load(c):
        return pltpu.make_async_copy(bank_ref.at[:, pl.ds(c * blk, blk)],
                                     dense_v.at[c % nbuf], dsem.at[c % nbuf])

    def mk_store(c, dst_ref, sem):
        return pltpu.make_async_copy(dense_v.at[c % nbuf],
                                     dst_ref.at[:, pl.ds(c * blk, blk)],
                                     sem.at[c % nbuf])

    lag = nbuf - 2
    for c in range(nb):
        if c >= nbuf:
            mk_store(c - nbuf, pre_ref, dsem_pre).wait()
            mk_store(c - nbuf, new_ref, dsem_new).wait()
        mk_load(c).start()
        if c >= lag:
            d = c - lag
            mk_load(d).wait()
            mk_store(d, pre_ref, dsem_pre).start()
            mk_store(d, new_ref, dsem_new).start()
    for d in range(nb - lag, nb):
        mk_load(d).wait()
        mk_store(d, pre_ref, dsem_pre).start()
        mk_store(d, new_ref, dsem_new).start()
    dense_tail = min(nbuf, nb)

    cp_ext.wait()
    for h, (k, s, ld) in enumerate(hits):
        ld.wait()
        q = lax.rem(k * blk - p + size, size)
        out_v[h] = _select_block(q, in_v[h], ext_v, blk, size, bs)

    for d in range(nb - dense_tail, nb):
        mk_store(d, pre_ref, dsem_pre).wait()
        mk_store(d, new_ref, dsem_new).wait()
    stores = []
    for h, (k, s, ld) in enumerate(hits):
        st = pltpu.make_async_copy(out_v.at[h], new_ref.at[:, pl.ds(s, blk)],
                                   hsems.at[h])
        st.start()
        stores.append(st)
    for st in stores:
        st.wait()


def _tc_bank(output, bank, ptr_arr):
    dim, size = bank.shape
    bs = output.shape[0]
    blk = 4096
    nhit = (bs - 1) // blk + 2
    w = bs + 2 * blk + 128
    nbuf = 12

    out_t = output.T.astype(jnp.float32)
    zpad = jnp.zeros((dim, blk), jnp.float32)
    zpad_r = jnp.zeros((dim, blk + 128), jnp.float32)
    ext = jnp.concatenate([zpad, out_t, zpad_r], axis=1)

    out_shapes = (
        jax.ShapeDtypeStruct((dim, size), jnp.float32),
        jax.ShapeDtypeStruct((dim, size), jnp.float32),
    )
    any_spec = pl.BlockSpec(memory_space=pl.ANY)
    fn = pl.pallas_call(
        functools.partial(_tc_body, blk=blk, size=size, bs=bs, nhit=nhit,
                          nbuf=nbuf),
        in_specs=[
            pl.BlockSpec(memory_space=pltpu.SMEM),
            any_spec, any_spec,
        ],
        out_specs=[any_spec, any_spec],
        out_shape=out_shapes,
        scratch_shapes=[
            pltpu.VMEM((dim, w), jnp.float32),
            pltpu.VMEM((nhit, dim, blk), jnp.float32),
            pltpu.VMEM((nhit, dim, blk), jnp.float32),
            pltpu.SemaphoreType.DMA((2,)),
            pltpu.SemaphoreType.DMA((2 * nhit,)),
            pltpu.VMEM((nbuf, dim, blk), jnp.float32),
            pltpu.SemaphoreType.DMA((nbuf,)),
            pltpu.SemaphoreType.DMA((nbuf,)),
            pltpu.SemaphoreType.DMA((nbuf,)),
        ],
    )
    return fn(ptr_arr, bank, ext)


def _sc_labels_body(ptr_hbm, bl_hbm, lab_hbm, lpre_hbm, lnew_hbm,
                    buf_v, lab_v, ptr_v, *, size, bs):
    nw = _NC * _NS
    sl = size // nw
    wid = lax.axis_index("s") * _NC + lax.axis_index("c")
    base = wid * sl

    pltpu.sync_copy(ptr_hbm, ptr_v)
    pltpu.sync_copy(lab_hbm, lab_v)
    pltpu.sync_copy(bl_hbm.at[pl.ds(base, sl)], buf_v)
    pltpu.sync_copy(buf_v, lpre_hbm.at[pl.ds(base, sl)])

    p_vec = ptr_v[...]
    base_vec = jnp.full((_LANES,), base, jnp.int32)

    def step(t, carry):
        i_vec = lax.iota(jnp.int32, _LANES) + t * _LANES
        d = lax.rem(p_vec + i_vec, size)
        local = d - base_vec
        m = jnp.logical_and(local >= 0, local < sl)
        vals = lab_v[pl.ds(t * _LANES, _LANES)]
        plsc.store_scatter(buf_v, [jnp.clip(local, 0, sl - 1)], vals, mask=m)
        return carry

    p = lax.reduce_max(p_vec, (0,))
    r1lo = jnp.clip(base - p, 0, bs)
    r1hi = jnp.clip(base + sl - p, 0, bs)
    r2lo = jnp.clip(base + size - p, 0, bs)
    r2hi = jnp.clip(base + sl + size - p, 0, bs)
    lax.fori_loop(lax.div(r1lo, _LANES),
                  lax.div(r1hi + _LANES - 1, _LANES), step, 0)
    lax.fori_loop(lax.div(r2lo, _LANES),
                  lax.div(r2hi + _LANES - 1, _LANES), step, 0)
    pltpu.sync_copy(buf_v, lnew_hbm.at[pl.ds(base, sl)])


def _sc_labels(labels, bank_labels, ptr_arr, size, bs):
    mesh = plsc.VectorSubcoreMesh(core_axis_name="c", subcore_axis_name="s",
                                  num_cores=_NC, num_subcores=_NS)
    sl = size // (_NC * _NS)
    fn = pl.kernel(
        functools.partial(_sc_labels_body, size=size, bs=bs),
        out_type=(
            jax.ShapeDtypeStruct((size,), jnp.int32),
            jax.ShapeDtypeStruct((size,), jnp.int32),
        ),
        mesh=mesh,
        compiler_params=pltpu.CompilerParams(needs_layout_passes=False, skip_device_barrier=True),
        scratch_types=[
            pltpu.VMEM((sl,), jnp.int32),
            pltpu.VMEM((bs,), jnp.int32),
            pltpu.VMEM((_LANES,), jnp.int32),
        ],
    )
    ptr16 = jnp.broadcast_to(ptr_arr, (_LANES,))
    return fn(ptr16, bank_labels, labels)


def kernel(output, labels, bank, bank_labels, ptr):
    dim, size = bank.shape
    bs = output.shape[0]
    ptr_arr = jnp.asarray(ptr, jnp.int32).reshape(1)

    lpre, lnew = _sc_labels(labels, bank_labels, ptr_arr, size, bs)
    pre, new = _tc_bank(output, bank, ptr_arr)
    return (output, pre, lpre, new, lnew)


# speedup vs baseline: 1.0302x; 1.0085x over previous
import functools

import jax
import jax.numpy as jnp
from jax import lax
from jax.experimental import pallas as pl
from jax.experimental.pallas import tpu as pltpu
from jax.experimental.pallas import tpu_sc as plsc

_NC = 1
_NS = 16
_LANES = 16


def _dyn_slice(ref, start, blk):
    wsl = blk + 128
    fine = lax.rem(start, 128)
    coarse = pl.multiple_of(start - fine, 128)
    win = ref[:, pl.ds(coarse, wsl)]
    rolled = pltpu.roll(win, lax.rem(wsl - fine, wsl), axis=1)
    return rolled[:, :blk]


def _select_block(q, bank_blk, ext, blk, size, bs):
    col = lax.broadcasted_iota(jnp.int32, (1, blk), 1)
    off = q + col
    wrapped = off >= size
    offm = jnp.where(wrapped, off - size, off)
    mask = offm < bs
    start1 = blk + jnp.minimum(q, bs)
    start2 = jnp.clip(q - (size - blk), 0, blk)
    part1 = _dyn_slice(ext, start1, blk)
    part2 = _dyn_slice(ext, start2, blk)
    g = jnp.where(wrapped, part2, part1)
    return jnp.where(mask, g, bank_blk)


def _tc_body(ptr_ref, bank_ref, ext_ref, pre_ref, new_ref,
             ext_v, in_v, out_v, sems, hsems,
             dense_v, dsem, dsem_pre, dsem_new,
             *, blk, size, bs, nhit, nbuf):
    p = ptr_ref[0]
    nb = size // blk

    cp_ext = pltpu.make_async_copy(ext_ref, ext_v, sems.at[0])
    cp_ext.start()

    k0 = lax.div(p, blk)
    hits = []
    for h in range(nhit):
        k = lax.rem(k0 + h, nb)
        s = pl.multiple_of(k * blk, blk)
        ld = pltpu.make_async_copy(bank_ref.at[:, pl.ds(s, blk)],
                                   in_v.at[h], hsems.at[h])
        ld.start()
        hits.append((k, s, ld))

    def mk_load(c):
        return pltpu.make_async_copy(bank_ref.at[:, pl.ds(c * blk, blk)],
                                     dense_v.at[c % nbuf], dsem.at[c % nbuf])

    def mk_store(c, dst_ref, sem):
        return pltpu.make_async_copy(dense_v.at[c % nbuf],
                                     dst_ref.at[:, pl.ds(c * blk, blk)],
                                     sem.at[c % nbuf])

    lag = nbuf - 2
    for c in range(nb):
        if c >= nbuf:
            mk_store(c - nbuf, pre_ref, dsem_pre).wait()
            mk_store(c - nbuf, new_ref, dsem_new).wait()
        mk_load(c).start()
        if c >= lag:
            d = c - lag
            mk_load(d).wait()
            mk_store(d, pre_ref, dsem_pre).start()
            mk_store(d, new_ref, dsem_new).start()
    for d in range(nb - lag, nb):
        mk_load(d).wait()
        mk_store(d, pre_ref, dsem_pre).start()
        mk_store(d, new_ref, dsem_new).start()
    dense_tail = min(nbuf, nb)

    cp_ext.wait()
    for h, (k, s, ld) in enumerate(hits):
        ld.wait()
        q = lax.rem(k * blk - p + size, size)
        out_v[h] = _select_block(q, in_v[h], ext_v, blk, size, bs)

    for d in range(nb - dense_tail, nb):
        mk_store(d, pre_ref, dsem_pre).wait()
        mk_store(d, new_ref, dsem_new).wait()
    stores = []
    for h, (k, s, ld) in enumerate(hits):
        st = pltpu.make_async_copy(out_v.at[h], new_ref.at[:, pl.ds(s, blk)],
                                   hsems.at[h])
        st.start()
        stores.append(st)
    for st in stores:
        st.wait()


def _tc_bank(output, bank, ptr_arr):
    dim, size = bank.shape
    bs = output.shape[0]
    blk = 4096
    nhit = (bs - 1) // blk + 2
    w = bs + 2 * blk + 128
    nbuf = 16

    out_t = output.T.astype(jnp.float32)
    zpad = jnp.zeros((dim, blk), jnp.float32)
    zpad_r = jnp.zeros((dim, blk + 128), jnp.float32)
    ext = jnp.concatenate([zpad, out_t, zpad_r], axis=1)

    out_shapes = (
        jax.ShapeDtypeStruct((dim, size), jnp.float32),
        jax.ShapeDtypeStruct((dim, size), jnp.float32),
    )
    any_spec = pl.BlockSpec(memory_space=pl.ANY)
    fn = pl.pallas_call(
        functools.partial(_tc_body, blk=blk, size=size, bs=bs, nhit=nhit,
                          nbuf=nbuf),
        in_specs=[
            pl.BlockSpec(memory_space=pltpu.SMEM),
            any_spec, any_spec,
        ],
        out_specs=[any_spec, any_spec],
        out_shape=out_shapes,
        scratch_shapes=[
            pltpu.VMEM((dim, w), jnp.float32),
            pltpu.VMEM((nhit, dim, blk), jnp.float32),
            pltpu.VMEM((nhit, dim, blk), jnp.float32),
            pltpu.SemaphoreType.DMA((2,)),
            pltpu.SemaphoreType.DMA((2 * nhit,)),
            pltpu.VMEM((nbuf, dim, blk), jnp.float32),
            pltpu.SemaphoreType.DMA((nbuf,)),
            pltpu.SemaphoreType.DMA((nbuf,)),
            pltpu.SemaphoreType.DMA((nbuf,)),
        ],
    )
    return fn(ptr_arr, bank, ext)


def _sc_labels_body(ptr_hbm, bl_hbm, lab_hbm, lpre_hbm, lnew_hbm,
                    buf_v, lab_v, ptr_v, *, size, bs):
    nw = _NC * _NS
    sl = size // nw
    wid = lax.axis_index("s") * _NC + lax.axis_index("c")
    base = wid * sl

    pltpu.sync_copy(ptr_hbm, ptr_v)
    pltpu.sync_copy(lab_hbm, lab_v)
    pltpu.sync_copy(bl_hbm.at[pl.ds(base, sl)], buf_v)
    pltpu.sync_copy(buf_v, lpre_hbm.at[pl.ds(base, sl)])

    p_vec = ptr_v[...]
    base_vec = jnp.full((_LANES,), base, jnp.int32)

    def step(t, carry):
        i_vec = lax.iota(jnp.int32, _LANES) + t * _LANES
        d = lax.rem(p_vec + i_vec, size)
        local = d - base_vec
        m = jnp.logical_and(local >= 0, local < sl)
        vals = lab_v[pl.ds(t * _LANES, _LANES)]
        plsc.store_scatter(buf_v, [jnp.clip(local, 0, sl - 1)], vals, mask=m)
        return carry

    p = lax.reduce_max(p_vec, (0,))
    r1lo = jnp.clip(base - p, 0, bs)
    r1hi = jnp.clip(base + sl - p, 0, bs)
    r2lo = jnp.clip(base + size - p, 0, bs)
    r2hi = jnp.clip(base + sl + size - p, 0, bs)
    lax.fori_loop(lax.div(r1lo, _LANES),
                  lax.div(r1hi + _LANES - 1, _LANES), step, 0)
    lax.fori_loop(lax.div(r2lo, _LANES),
                  lax.div(r2hi + _LANES - 1, _LANES), step, 0)
    pltpu.sync_copy(buf_v, lnew_hbm.at[pl.ds(base, sl)])


def _sc_labels(labels, bank_labels, ptr_arr, size, bs):
    mesh = plsc.VectorSubcoreMesh(core_axis_name="c", subcore_axis_name="s",
                                  num_cores=_NC, num_subcores=_NS)
    sl = size // (_NC * _NS)
    fn = pl.kernel(
        functools.partial(_sc_labels_body, size=size, bs=bs),
        out_type=(
            jax.ShapeDtypeStruct((size,), jnp.int32),
            jax.ShapeDtypeStruct((size,), jnp.int32),
        ),
        mesh=mesh,
        compiler_params=pltpu.CompilerParams(needs_layout_passes=False, skip_device_barrier=True),
        scratch_types=[
            pltpu.VMEM((sl,), jnp.int32),
            pltpu.VMEM((bs,), jnp.int32),
            pltpu.VMEM((_LANES,), jnp.int32),
        ],
    )
    ptr16 = jnp.broadcast_to(ptr_arr, (_LANES,))
    return fn(ptr16, bank_labels, labels)


def kernel(output, labels, bank, bank_labels, ptr):
    dim, size = bank.shape
    bs = output.shape[0]
    ptr_arr = jnp.asarray(ptr, jnp.int32).reshape(1)

    lpre, lnew = _sc_labels(labels, bank_labels, ptr_arr, size, bs)
    pre, new = _tc_bank(output, bank, ptr_arr)
    return (output, pre, lpre, new, lnew)
